# Initial kernel scaffold; baseline (speedup 1.0000x reference)
import jax
import jax.numpy as jnp
from jax.experimental import pallas as pl


def kernel(x, filter_coeff, edge_index, weight, bias):
    raise NotImplementedError("write your pallas kernel here")


# speedup vs baseline: 9.3719x; 9.3719x over previous
import functools

import jax
import jax.numpy as jnp
from jax import lax
from jax.experimental import pallas as pl
from jax.experimental.pallas import tpu as pltpu
from jax.experimental.pallas import tpu_sc as plsc

N = 10000
D = 128
E = 320000
K = 3
DW = 128

_info = plsc.get_sparse_core_info()
NC = _info.num_cores
NS = _info.num_subcores
NW = NC * NS
EW = E // NW
C = 80
NCHUNK = EW // C
NP = 10240
RPT = NP // NS
ZB = 128

_MESH = plsc.VectorSubcoreMesh(core_axis_name="c", subcore_axis_name="s")


def _deg_body(dst_hbm, out_hbm, dst_v, ones_v, zb_v, acc_sh):
    c = lax.axis_index("c")
    s = lax.axis_index("s")
    wid = s * NC + c
    one16 = jnp.full((16,), 1.0, jnp.float32)
    zero16 = jnp.zeros((16,), jnp.float32)

    def fill_ones(i, _):
        for j in range(DW // 16):
            ones_v[i, pl.ds(j * 16, 16)] = one16
        return 0

    lax.fori_loop(0, C, fill_ones, 0)

    def fill_zero(i, _):
        for j in range(DW // 16):
            zb_v[i, pl.ds(j * 16, 16)] = zero16
        return 0

    lax.fori_loop(0, ZB, fill_zero, 0)

    def zcp(i, _):
        pltpu.sync_copy(zb_v, acc_sh.at[pl.ds(s * RPT + i * ZB, ZB)])
        return 0

    lax.fori_loop(0, RPT // ZB, zcp, 0)
    plsc.subcore_barrier()

    ebase = wid * EW

    def step(i, _):
        pltpu.sync_copy(dst_hbm.at[pl.ds(ebase + i * C, C)], dst_v)
        pltpu.sync_copy(ones_v, acc_sh.at[dst_v], add=True)
        return 0

    lax.fori_loop(0, NCHUNK, step, 0)
    plsc.subcore_barrier()
    pltpu.sync_copy(acc_sh.at[pl.ds(s * RPT, RPT)],
                    out_hbm.at[c, pl.ds(s * RPT, RPT)])


_deg_call = functools.partial(
    pl.kernel,
    mesh=_MESH,
    out_type=jax.ShapeDtypeStruct((NC, NP, DW), jnp.float32),
    scratch_types=[
        pltpu.VMEM((C,), jnp.int32),
        pltpu.VMEM((C, DW), jnp.float32),
        pltpu.VMEM((ZB, DW), jnp.float32),
        pltpu.VMEM_SHARED((NP, DW), jnp.float32),
    ],
)(_deg_body)


def _spmv_body(h_hbm, src_hbm, dst_hbm, out_hbm,
               src_v, dst_v, rows_v, zb_v, acc_sh, sem):
    c = lax.axis_index("c")
    s = lax.axis_index("s")
    wid = s * NC + c
    zero16 = jnp.zeros((16,), jnp.float32)

    def fz(i, _):
        for j in range(D // 16):
            zb_v[i, pl.ds(j * 16, 16)] = zero16
        return 0

    lax.fori_loop(0, ZB, fz, 0)

    def zcp(i, _):
        pltpu.sync_copy(zb_v, acc_sh.at[pl.ds(s * RPT + i * ZB, ZB)])
        return 0

    lax.fori_loop(0, RPT // ZB, zcp, 0)
    plsc.subcore_barrier()

    ebase = wid * EW

    def step(i, _):
        eb = ebase + i * C
        pltpu.sync_copy(src_hbm.at[pl.ds(eb, C)], src_v)
        pltpu.sync_copy(dst_hbm.at[pl.ds(eb, C)], dst_v)
        pltpu.async_copy(h_hbm.at[src_v], rows_v, sem).wait()
        pltpu.sync_copy(rows_v, acc_sh.at[dst_v], add=True)
        return 0

    lax.fori_loop(0, NCHUNK, step, 0)
    plsc.subcore_barrier()
    pltpu.sync_copy(acc_sh.at[pl.ds(s * RPT, RPT)],
                    out_hbm.at[c, pl.ds(s * RPT, RPT)])


_spmv_call = functools.partial(
    pl.kernel,
    mesh=_MESH,
    out_type=jax.ShapeDtypeStruct((NC, NP, D), jnp.float32),
    scratch_types=[
        pltpu.VMEM((C,), jnp.int32),
        pltpu.VMEM((C,), jnp.int32),
        pltpu.VMEM((C, D), jnp.float32),
        pltpu.VMEM((ZB, D), jnp.float32),
        pltpu.VMEM_SHARED((NP, D), jnp.float32),
        pltpu.SemaphoreType.DMA,
    ],
)(_spmv_body)


BN = 1000


def _scale1_body(degp_ref, x_ref, dinv_ref, h0_ref):
    deg = degp_ref[0, :, 0:1] + degp_ref[1, :, 0:1]
    dinv = lax.rsqrt(jnp.maximum(deg, 1.0))
    dinv_ref[...] = dinv
    h0_ref[...] = x_ref[...] * dinv


def _scale1_call(degp, x):
    return pl.pallas_call(
        _scale1_body,
        grid=(N // BN,),
        in_specs=[
            pl.BlockSpec((NC, BN, DW), lambda i: (0, i, 0)),
            pl.BlockSpec((BN, D), lambda i: (i, 0)),
        ],
        out_specs=[
            pl.BlockSpec((BN, 1), lambda i: (i, 0)),
            pl.BlockSpec((BN, D), lambda i: (i, 0)),
        ],
        out_shape=[
            jax.ShapeDtypeStruct((N, 1), jnp.float32),
            jax.ShapeDtypeStruct((N, D), jnp.float32),
        ],
    )(degp, x)


def _scale2_body(p_ref, dinv_ref, tx1_ref, h1_ref):
    dinv = dinv_ref[...]
    tx1 = -(dinv * (p_ref[0] + p_ref[1]))
    tx1_ref[...] = tx1
    h1_ref[...] = dinv * tx1


def _scale2_call(p, dinv):
    return pl.pallas_call(
        _scale2_body,
        grid=(N // BN,),
        in_specs=[
            pl.BlockSpec((NC, BN, D), lambda i: (0, i, 0)),
            pl.BlockSpec((BN, 1), lambda i: (i, 0)),
        ],
        out_specs=[
            pl.BlockSpec((BN, D), lambda i: (i, 0)),
            pl.BlockSpec((BN, D), lambda i: (i, 0)),
        ],
        out_shape=[
            jax.ShapeDtypeStruct((N, D), jnp.float32),
            jax.ShapeDtypeStruct((N, D), jnp.float32),
        ],
    )(p, dinv)


def _final_body(x_ref, tx1_ref, q_ref, dinv_ref, fc_ref, w_ref, b_ref, out_ref):
    x = x_ref[...]
    tx1 = tx1_ref[...]
    tx2 = -2.0 * dinv_ref[...] * (q_ref[0] + q_ref[1]) - x
    fc = fc_ref[...]
    acc = jnp.dot(fc[:, 0:1] * x, w_ref[0], preferred_element_type=jnp.float32)
    acc = acc + jnp.dot(fc[:, 1:2] * tx1, w_ref[1],
                        preferred_element_type=jnp.float32)
    acc = acc + jnp.dot(fc[:, 2:3] * tx2, w_ref[2],
                        preferred_element_type=jnp.float32)
    out_ref[...] = acc + b_ref[...]


def _final_call(x, tx1, q, dinv, fc_t, weight, bias2d):
    return pl.pallas_call(
        _final_body,
        grid=(N // BN,),
        in_specs=[
            pl.BlockSpec((BN, D), lambda i: (i, 0)),
            pl.BlockSpec((BN, D), lambda i: (i, 0)),
            pl.BlockSpec((NC, BN, D), lambda i: (0, i, 0)),
            pl.BlockSpec((BN, 1), lambda i: (i, 0)),
            pl.BlockSpec((BN, K), lambda i: (i, 0)),
            pl.BlockSpec((K, D, D), lambda i: (0, 0, 0)),
            pl.BlockSpec((1, D), lambda i: (0, 0)),
        ],
        out_specs=pl.BlockSpec((BN, D), lambda i: (i, 0)),
        out_shape=jax.ShapeDtypeStruct((N, D), jnp.float32),
    )(x, tx1, q, dinv, fc_t, weight, bias2d)


def kernel(x, filter_coeff, edge_index, weight, bias):
    src = edge_index[0]
    dst = edge_index[1]
    fc_t = jnp.transpose(filter_coeff[:, :, 0])
    bias2d = bias.reshape(1, D)

    degp = _deg_call(dst)
    dinv, h0 = _scale1_call(degp, x)
    p = _spmv_call(h0, src, dst)
    tx1, h1 = _scale2_call(p, dinv)
    q = _spmv_call(h1, src, dst)
    return _final_call(x, tx1, q, dinv, fc_t, weight, bias2d)


# speedup vs baseline: 23.3934x; 2.4961x over previous
import functools

import jax
import jax.numpy as jnp
from jax import lax
from jax.experimental import pallas as pl
from jax.experimental.pallas import tpu as pltpu
from jax.experimental.pallas import tpu_sc as plsc

N = 10000
D = 128
E = 320000
K = 3
DW = 128

_info = plsc.get_sparse_core_info()
NC = _info.num_cores
NS = _info.num_subcores
NW = NC * NS
EW = E // NW
C = 40
NCHUNK = EW // C
NBUF = 5
NGRP = NCHUNK // NBUF
CD = 80
DCHUNK = EW // CD
DBUF = 5
DGRP = DCHUNK // DBUF
NP = 10240
RPT = NP // NS
ZB = 128

_MESH = plsc.VectorSubcoreMesh(core_axis_name="c", subcore_axis_name="s")


def _deg_body(dst_hbm, out_hbm, dstr_v, ones_v, zb_v, acc_sh,
              i0, i1, i2, i3, i4, t0, t1, t2, t3, t4):
    c = lax.axis_index("c")
    s = lax.axis_index("s")
    wid = s * NC + c
    isems = (i0, i1, i2, i3, i4)
    ssems = (t0, t1, t2, t3, t4)
    one16 = jnp.full((16,), 1.0, jnp.float32)
    zero16 = jnp.zeros((16,), jnp.float32)
    ebase = wid * EW

    def fill_ones(i, _):
        for j in range(DW // 16):
            ones_v[i, pl.ds(j * 16, 16)] = one16
        return 0

    lax.fori_loop(0, CD, fill_ones, 0)

    def fill_zero(i, _):
        for j in range(DW // 16):
            zb_v[i, pl.ds(j * 16, 16)] = zero16
        return 0

    lax.fori_loop(0, ZB, fill_zero, 0)

    def zcp(i, _):
        pltpu.sync_copy(zb_v, acc_sh.at[pl.ds(s * RPT + i * ZB, ZB)])
        return 0

    lax.fori_loop(0, RPT // ZB, zcp, 0)
    plsc.subcore_barrier()

    for j in range(DBUF):
        pltpu.async_copy(dst_hbm.at[pl.ds(ebase + j * CD, CD)],
                         dstr_v.at[j], isems[j])

    def grp(g, _):
        for j in range(DBUF):
            i = g * DBUF + j
            pltpu.make_async_copy(dst_hbm.at[pl.ds(ebase, CD)],
                                  dstr_v.at[j], isems[j]).wait()
            pltpu.async_copy(ones_v, acc_sh.at[dstr_v.at[j]], ssems[j],
                             add=True)
            pltpu.make_async_copy(ones_v, acc_sh.at[dstr_v.at[j]],
                                  ssems[j]).wait()
            pltpu.async_copy(dst_hbm.at[pl.ds(ebase + (i + DBUF) * CD, CD)],
                             dstr_v.at[j], isems[j])
        return 0

    lax.fori_loop(0, DGRP - 1, grp, 0)
    for j in range(DBUF):
        pltpu.make_async_copy(dst_hbm.at[pl.ds(ebase, CD)],
                              dstr_v.at[j], isems[j]).wait()
        pltpu.async_copy(ones_v, acc_sh.at[dstr_v.at[j]], ssems[j], add=True)
        pltpu.make_async_copy(ones_v, acc_sh.at[dstr_v.at[j]],
                              ssems[j]).wait()

    plsc.subcore_barrier()
    pltpu.sync_copy(acc_sh.at[pl.ds(s * RPT, RPT)],
                    out_hbm.at[c, pl.ds(s * RPT, RPT)])


_deg_call = functools.partial(
    pl.kernel,
    mesh=_MESH,
    out_type=jax.ShapeDtypeStruct((NC, NP, DW), jnp.float32),
    scratch_types=[
        pltpu.VMEM((DBUF, CD), jnp.int32),
        pltpu.VMEM((CD, DW), jnp.float32),
        pltpu.VMEM((ZB, DW), jnp.float32),
        pltpu.VMEM_SHARED((NP, DW), jnp.float32),
        pltpu.SemaphoreType.DMA,
        pltpu.SemaphoreType.DMA,
        pltpu.SemaphoreType.DMA,
        pltpu.SemaphoreType.DMA,
        pltpu.SemaphoreType.DMA,
        pltpu.SemaphoreType.DMA,
        pltpu.SemaphoreType.DMA,
        pltpu.SemaphoreType.DMA,
        pltpu.SemaphoreType.DMA,
        pltpu.SemaphoreType.DMA,
    ],
)(_deg_body)


def _spmv_body(h_hbm, src_hbm, dst_hbm, out_hbm,
               src_v, dstr_v, rows_v, acc_sh,
               g0, g1, g2, g3, g4, i0, i1, i2, i3, i4):
    c = lax.axis_index("c")
    s = lax.axis_index("s")
    wid = s * NC + c
    gsems = (g0, g1, g2, g3, g4)
    isems = (i0, i1, i2, i3, i4)
    zero16 = jnp.zeros((16,), jnp.float32)
    ebase = wid * EW

    pltpu.sync_copy(src_hbm.at[pl.ds(ebase, EW)], src_v)

    def fz(i, _):
        for b in range(NBUF):
            for j in range(D // 16):
                rows_v[b, i, pl.ds(j * 16, 16)] = zero16
        return 0

    lax.fori_loop(0, C, fz, 0)

    def zcp(i, _):
        pltpu.sync_copy(rows_v.at[0], acc_sh.at[pl.ds(s * RPT + i * C, C)])
        return 0

    lax.fori_loop(0, RPT // C, zcp, 0)
    plsc.subcore_barrier()

    for j in range(NBUF):
        pltpu.async_copy(h_hbm.at[src_v.at[pl.ds(j * C, C)]], rows_v.at[j],
                         gsems[j])
        pltpu.async_copy(dst_hbm.at[pl.ds(ebase + j * C, C)], dstr_v.at[j],
                         isems[j])

    def grp(g, _):
        for j in range(NBUF):
            i = g * NBUF + j
            pltpu.make_async_copy(h_hbm.at[src_v.at[pl.ds(0, C)]],
                                  rows_v.at[j], gsems[j]).wait()
            pltpu.make_async_copy(dst_hbm.at[pl.ds(ebase, C)],
                                  dstr_v.at[j], isems[j]).wait()
            pltpu.sync_copy(rows_v.at[j], acc_sh.at[dstr_v.at[j]], add=True)
            pltpu.async_copy(h_hbm.at[src_v.at[pl.ds((i + NBUF) * C, C)]],
                             rows_v.at[j], gsems[j])
            pltpu.async_copy(dst_hbm.at[pl.ds(ebase + (i + NBUF) * C, C)],
                             dstr_v.at[j], isems[j])
        return 0

    lax.fori_loop(0, NGRP - 1, grp, 0)
    for j in range(NBUF):
        pltpu.make_async_copy(h_hbm.at[src_v.at[pl.ds(0, C)]],
                              rows_v.at[j], gsems[j]).wait()
        pltpu.make_async_copy(dst_hbm.at[pl.ds(ebase, C)],
                              dstr_v.at[j], isems[j]).wait()
        pltpu.sync_copy(rows_v.at[j], acc_sh.at[dstr_v.at[j]], add=True)

    plsc.subcore_barrier()
    pltpu.sync_copy(acc_sh.at[pl.ds(s * RPT, RPT)],
                    out_hbm.at[c, pl.ds(s * RPT, RPT)])


_spmv_call = functools.partial(
    pl.kernel,
    mesh=_MESH,
    out_type=jax.ShapeDtypeStruct((NC, NP, D), jnp.float32),
    scratch_types=[
        pltpu.VMEM((EW,), jnp.int32),
        pltpu.VMEM((NBUF, C), jnp.int32),
        pltpu.VMEM((NBUF, C, D), jnp.float32),
        pltpu.VMEM_SHARED((NP, D), jnp.float32),
        pltpu.SemaphoreType.DMA,
        pltpu.SemaphoreType.DMA,
        pltpu.SemaphoreType.DMA,
        pltpu.SemaphoreType.DMA,
        pltpu.SemaphoreType.DMA,
        pltpu.SemaphoreType.DMA,
        pltpu.SemaphoreType.DMA,
        pltpu.SemaphoreType.DMA,
        pltpu.SemaphoreType.DMA,
        pltpu.SemaphoreType.DMA,
    ],
)(_spmv_body)


BN = 1000


def _scale1_body(degp_ref, x_ref, dinv_ref, h0_ref):
    deg = degp_ref[0, :, 0:1] + degp_ref[1, :, 0:1]
    dinv = lax.rsqrt(jnp.maximum(deg, 1.0))
    dinv_ref[...] = dinv
    h0_ref[...] = x_ref[...] * dinv


def _scale1_call(degp, x):
    return pl.pallas_call(
        _scale1_body,
        grid=(N // BN,),
        in_specs=[
            pl.BlockSpec((NC, BN, DW), lambda i: (0, i, 0)),
            pl.BlockSpec((BN, D), lambda i: (i, 0)),
        ],
        out_specs=[
            pl.BlockSpec((BN, 1), lambda i: (i, 0)),
            pl.BlockSpec((BN, D), lambda i: (i, 0)),
        ],
        out_shape=[
            jax.ShapeDtypeStruct((N, 1), jnp.float32),
            jax.ShapeDtypeStruct((N, D), jnp.float32),
        ],
    )(degp, x)


def _scale2_body(p_ref, dinv_ref, tx1_ref, h1_ref):
    dinv = dinv_ref[...]
    tx1 = -(dinv * (p_ref[0] + p_ref[1]))
    tx1_ref[...] = tx1
    h1_ref[...] = dinv * tx1


def _scale2_call(p, dinv):
    return pl.pallas_call(
        _scale2_body,
        grid=(N // BN,),
        in_specs=[
            pl.BlockSpec((NC, BN, D), lambda i: (0, i, 0)),
            pl.BlockSpec((BN, 1), lambda i: (i, 0)),
        ],
        out_specs=[
            pl.BlockSpec((BN, D), lambda i: (i, 0)),
            pl.BlockSpec((BN, D), lambda i: (i, 0)),
        ],
        out_shape=[
            jax.ShapeDtypeStruct((N, D), jnp.float32),
            jax.ShapeDtypeStruct((N, D), jnp.float32),
        ],
    )(p, dinv)


def _final_body(x_ref, tx1_ref, q_ref, dinv_ref, fc_ref, w_ref, b_ref, out_ref):
    x = x_ref[...]
    tx1 = tx1_ref[...]
    tx2 = -2.0 * dinv_ref[...] * (q_ref[0] + q_ref[1]) - x
    fc = fc_ref[...]
    acc = jnp.dot(fc[:, 0:1] * x, w_ref[0], preferred_element_type=jnp.float32)
    acc = acc + jnp.dot(fc[:, 1:2] * tx1, w_ref[1],
                        preferred_element_type=jnp.float32)
    acc = acc + jnp.dot(fc[:, 2:3] * tx2, w_ref[2],
                        preferred_element_type=jnp.float32)
    out_ref[...] = acc + b_ref[...]


def _final_call(x, tx1, q, dinv, fc_t, weight, bias2d):
    return pl.pallas_call(
        _final_body,
        grid=(N // BN,),
        in_specs=[
            pl.BlockSpec((BN, D), lambda i: (i, 0)),
            pl.BlockSpec((BN, D), lambda i: (i, 0)),
            pl.BlockSpec((NC, BN, D), lambda i: (0, i, 0)),
            pl.BlockSpec((BN, 1), lambda i: (i, 0)),
            pl.BlockSpec((BN, K), lambda i: (i, 0)),
            pl.BlockSpec((K, D, D), lambda i: (0, 0, 0)),
            pl.BlockSpec((1, D), lambda i: (0, 0)),
        ],
        out_specs=pl.BlockSpec((BN, D), lambda i: (i, 0)),
        out_shape=jax.ShapeDtypeStruct((N, D), jnp.float32),
    )(x, tx1, q, dinv, fc_t, weight, bias2d)


def kernel(x, filter_coeff, edge_index, weight, bias):
    src = edge_index[0]
    dst = edge_index[1]
    fc_t = jnp.transpose(filter_coeff[:, :, 0])
    bias2d = bias.reshape(1, D)

    degp = _deg_call(dst)
    dinv, h0 = _scale1_call(degp, x)
    p = _spmv_call(h0, src, dst)
    tx1, h1 = _scale2_call(p, dinv)
    q = _spmv_call(h1, src, dst)
    return _final_call(x, tx1, q, dinv, fc_t, weight, bias2d)


# speedup vs baseline: 26.5594x; 1.1353x over previous
import functools

import jax
import jax.numpy as jnp
from jax import lax
from jax.experimental import pallas as pl
from jax.experimental.pallas import tpu as pltpu
from jax.experimental.pallas import tpu_sc as plsc

N = 10000
D = 128
E = 320000
K = 3
DW = 128

_info = plsc.get_sparse_core_info()
NC = _info.num_cores
NS = _info.num_subcores
NW = NC * NS
EW = E // NW
C = 80
NCHUNK = EW // C
NBUF = 3
NSTD = 40
CD = 80
DCHUNK = EW // CD
DBUF = 5
DGRP = DCHUNK // DBUF
NP = 10240
RPT = NP // NS
ZB = 128

_MESH = plsc.VectorSubcoreMesh(core_axis_name="c", subcore_axis_name="s")


def _deg_body(dst_hbm, out_hbm, dstr_v, ones_v, zb_v, acc_sh,
              i0, i1, i2, i3, i4, t0, t1, t2, t3, t4):
    c = lax.axis_index("c")
    s = lax.axis_index("s")
    wid = s * NC + c
    isems = (i0, i1, i2, i3, i4)
    ssems = (t0, t1, t2, t3, t4)
    one16 = jnp.full((16,), 1.0, jnp.float32)
    zero16 = jnp.zeros((16,), jnp.float32)
    ebase = wid * EW

    def fill_ones(i, _):
        ones_v[pl.ds(i * 16, 16)] = one16
        return 0

    lax.fori_loop(0, CD // 16, fill_ones, 0)

    def fill_zero(i, _):
        zb_v[pl.ds(i * 16, 16)] = zero16
        return 0

    lax.fori_loop(0, RPT // 16, fill_zero, 0)
    pltpu.sync_copy(zb_v, acc_sh.at[pl.ds(s * RPT, RPT)])
    plsc.subcore_barrier()

    for j in range(DBUF):
        pltpu.async_copy(dst_hbm.at[pl.ds(ebase + j * CD, CD)],
                         dstr_v.at[j], isems[j])

    def grp(g, _):
        for j in range(DBUF):
            i = g * DBUF + j
            pltpu.make_async_copy(dst_hbm.at[pl.ds(ebase, CD)],
                                  dstr_v.at[j], isems[j]).wait()
            pltpu.async_copy(ones_v, acc_sh.at[dstr_v.at[j]], ssems[j],
                             add=True)
            pltpu.make_async_copy(ones_v, acc_sh.at[dstr_v.at[j]],
                                  ssems[j]).wait()
            pltpu.async_copy(dst_hbm.at[pl.ds(ebase + (i + DBUF) * CD, CD)],
                             dstr_v.at[j], isems[j])
        return 0

    lax.fori_loop(0, DGRP - 1, grp, 0)
    for j in range(DBUF):
        pltpu.make_async_copy(dst_hbm.at[pl.ds(ebase, CD)],
                              dstr_v.at[j], isems[j]).wait()
        pltpu.async_copy(ones_v, acc_sh.at[dstr_v.at[j]], ssems[j], add=True)
        pltpu.make_async_copy(ones_v, acc_sh.at[dstr_v.at[j]],
                              ssems[j]).wait()

    plsc.subcore_barrier()
    pltpu.sync_copy(acc_sh.at[pl.ds(s * RPT, RPT)],
                    out_hbm.at[c, pl.ds(s * RPT, RPT)])


_deg_call = functools.partial(
    pl.kernel,
    mesh=_MESH,
    out_type=jax.ShapeDtypeStruct((NC, NP), jnp.float32),
    scratch_types=[
        pltpu.VMEM((DBUF, CD), jnp.int32),
        pltpu.VMEM((CD,), jnp.float32),
        pltpu.VMEM((RPT,), jnp.float32),
        pltpu.VMEM_SHARED((NP,), jnp.float32),
        pltpu.SemaphoreType.DMA,
        pltpu.SemaphoreType.DMA,
        pltpu.SemaphoreType.DMA,
        pltpu.SemaphoreType.DMA,
        pltpu.SemaphoreType.DMA,
        pltpu.SemaphoreType.DMA,
        pltpu.SemaphoreType.DMA,
        pltpu.SemaphoreType.DMA,
        pltpu.SemaphoreType.DMA,
        pltpu.SemaphoreType.DMA,
    ],
)(_deg_body)


def _spmv_body(h_hbm, src_hbm, dst_hbm, out_hbm,
               src_v, dstr_v, rows_v, acc_sh,
               g0, g1, g2, g3, g4, i0, i1, i2, i3, i4):
    c = lax.axis_index("c")
    s = lax.axis_index("s")
    wid = s * NC + c
    gsems = (g0, g1, g2, g3, g4)
    isems = (i0, i1, i2, i3, i4)
    zero16 = jnp.zeros((16,), jnp.float32)
    ebase = wid * EW

    pltpu.sync_copy(src_hbm.at[pl.ds(ebase, EW)], src_v)

    def fz(i, _):
        for b in range(NBUF):
            for j in range(D // 16):
                rows_v[b, i, pl.ds(j * 16, 16)] = zero16
        return 0

    lax.fori_loop(0, C, fz, 0)

    def zcp(i, _):
        pltpu.sync_copy(rows_v.at[0], acc_sh.at[pl.ds(s * RPT + i * C, C)])
        return 0

    lax.fori_loop(0, RPT // C, zcp, 0)
    plsc.subcore_barrier()

    for j in range(NBUF):
        pltpu.async_copy(h_hbm.at[src_v.at[pl.ds(j * C, C)]], rows_v.at[j],
                         gsems[j])
        pltpu.async_copy(dst_hbm.at[pl.ds(ebase + j * C, C)], dstr_v.at[j],
                         isems[j])

    def consume(i, j):
        pltpu.make_async_copy(h_hbm.at[src_v.at[pl.ds(0, C)]],
                              rows_v.at[j], gsems[j]).wait()
        pltpu.make_async_copy(dst_hbm.at[pl.ds(ebase, C)],
                              dstr_v.at[j], isems[j]).wait()
        pltpu.sync_copy(rows_v.at[j], acc_sh.at[dstr_v.at[j]], add=True)

    def fire(i, j):
        pltpu.async_copy(h_hbm.at[src_v.at[pl.ds(i * C, C)]],
                         rows_v.at[j], gsems[j])
        pltpu.async_copy(dst_hbm.at[pl.ds(ebase + i * C, C)], dstr_v.at[j],
                         isems[j])

    def grp(g, _):
        for j in range(NBUF):
            i = g * NBUF + j
            consume(i, j)
            fire(i + NBUF, j)
        return 0

    lax.fori_loop(0, NSTD, grp, 0)
    consume(120, 0)
    fire(123, 0)
    consume(121, 1)
    fire(124, 1)
    consume(122, 2)
    consume(123, 0)
    consume(124, 1)

    plsc.subcore_barrier()
    pltpu.sync_copy(acc_sh.at[pl.ds(s * RPT, RPT)],
                    out_hbm.at[c, pl.ds(s * RPT, RPT)])


_spmv_call = functools.partial(
    pl.kernel,
    mesh=_MESH,
    out_type=jax.ShapeDtypeStruct((NC, NP, D), jnp.float32),
    scratch_types=[
        pltpu.VMEM((EW,), jnp.int32),
        pltpu.VMEM((NBUF, C), jnp.int32),
        pltpu.VMEM((NBUF, C, D), jnp.float32),
        pltpu.VMEM_SHARED((NP, D), jnp.float32),
        pltpu.SemaphoreType.DMA,
        pltpu.SemaphoreType.DMA,
        pltpu.SemaphoreType.DMA,
        pltpu.SemaphoreType.DMA,
        pltpu.SemaphoreType.DMA,
        pltpu.SemaphoreType.DMA,
        pltpu.SemaphoreType.DMA,
        pltpu.SemaphoreType.DMA,
        pltpu.SemaphoreType.DMA,
        pltpu.SemaphoreType.DMA,
    ],
)(_spmv_body)


BN = 1000


def _scale1_body(degp_ref, x_ref, dinv_ref, h0_ref):
    deg = degp_ref[0] + degp_ref[1]
    dinv = lax.rsqrt(jnp.maximum(deg, 1.0))
    dinv_ref[...] = dinv
    h0_ref[...] = x_ref[...] * dinv


def _scale1_call(degp, x):
    return pl.pallas_call(
        _scale1_body,
        grid=(N // BN,),
        in_specs=[
            pl.BlockSpec((NC, BN, 1), lambda i: (0, i, 0)),
            pl.BlockSpec((BN, D), lambda i: (i, 0)),
        ],
        out_specs=[
            pl.BlockSpec((BN, 1), lambda i: (i, 0)),
            pl.BlockSpec((BN, D), lambda i: (i, 0)),
        ],
        out_shape=[
            jax.ShapeDtypeStruct((N, 1), jnp.float32),
            jax.ShapeDtypeStruct((N, D), jnp.float32),
        ],
    )(degp, x)


def _scale2_body(p_ref, dinv_ref, tx1_ref, h1_ref):
    dinv = dinv_ref[...]
    tx1 = -(dinv * (p_ref[0] + p_ref[1]))
    tx1_ref[...] = tx1
    h1_ref[...] = dinv * tx1


def _scale2_call(p, dinv):
    return pl.pallas_call(
        _scale2_body,
        grid=(N // BN,),
        in_specs=[
            pl.BlockSpec((NC, BN, D), lambda i: (0, i, 0)),
            pl.BlockSpec((BN, 1), lambda i: (i, 0)),
        ],
        out_specs=[
            pl.BlockSpec((BN, D), lambda i: (i, 0)),
            pl.BlockSpec((BN, D), lambda i: (i, 0)),
        ],
        out_shape=[
            jax.ShapeDtypeStruct((N, D), jnp.float32),
            jax.ShapeDtypeStruct((N, D), jnp.float32),
        ],
    )(p, dinv)


def _final_body(x_ref, tx1_ref, q_ref, dinv_ref, fc_ref, w_ref, b_ref, out_ref):
    x = x_ref[...]
    tx1 = tx1_ref[...]
    tx2 = -2.0 * dinv_ref[...] * (q_ref[0] + q_ref[1]) - x
    fc = fc_ref[...]
    acc = jnp.dot(fc[:, 0:1] * x, w_ref[0], preferred_element_type=jnp.float32)
    acc = acc + jnp.dot(fc[:, 1:2] * tx1, w_ref[1],
                        preferred_element_type=jnp.float32)
    acc = acc + jnp.dot(fc[:, 2:3] * tx2, w_ref[2],
                        preferred_element_type=jnp.float32)
    out_ref[...] = acc + b_ref[...]


def _final_call(x, tx1, q, dinv, fc_t, weight, bias2d):
    return pl.pallas_call(
        _final_body,
        grid=(N // BN,),
        in_specs=[
            pl.BlockSpec((BN, D), lambda i: (i, 0)),
            pl.BlockSpec((BN, D), lambda i: (i, 0)),
            pl.BlockSpec((NC, BN, D), lambda i: (0, i, 0)),
            pl.BlockSpec((BN, 1), lambda i: (i, 0)),
            pl.BlockSpec((BN, K), lambda i: (i, 0)),
            pl.BlockSpec((K, D, D), lambda i: (0, 0, 0)),
            pl.BlockSpec((1, D), lambda i: (0, 0)),
        ],
        out_specs=pl.BlockSpec((BN, D), lambda i: (i, 0)),
        out_shape=jax.ShapeDtypeStruct((N, D), jnp.float32),
    )(x, tx1, q, dinv, fc_t, weight, bias2d)


def kernel(x, filter_coeff, edge_index, weight, bias):
    src = edge_index[0]
    dst = edge_index[1]
    fc_t = jnp.transpose(filter_coeff[:, :, 0])
    bias2d = bias.reshape(1, D)

    degp = _deg_call(dst).reshape(NC, NP, 1)
    dinv, h0 = _scale1_call(degp, x)
    p = _spmv_call(h0, src, dst)
    tx1, h1 = _scale2_call(p, dinv)
    q = _spmv_call(h1, src, dst)
    return _final_call(x, tx1, q, dinv, fc_t, weight, bias2d)
